# baseline (speedup 1.0000x reference)
import functools

import jax
import jax.numpy as jnp
from jax import lax
from jax.experimental import pallas as pl
from jax.experimental.pallas import tpu as pltpu
from jax.experimental.pallas import tpu_sc as plsc

VOCAB = 100000
EMBED = 128
PROJ = 512
LN_EPS = 1e-5

NC = 2
NS = 16
NW = NC * NS

LPAD = 56
CH_BATCHES = 2
BB = 16


def _make_sc_gather(n_rows, n_chunks, chunk):
    per_w = n_chunks * chunk
    mesh = plsc.VectorSubcoreMesh(core_axis_name="c", subcore_axis_name="s")

    @functools.partial(
        pl.kernel,
        out_type=jax.ShapeDtypeStruct((n_rows, EMBED), jnp.float32),
        mesh=mesh,
        scratch_types=[
            pltpu.VMEM((n_chunks, chunk), jnp.int32),
            pltpu.VMEM((chunk, EMBED), jnp.float32),
            pltpu.SemaphoreType.DMA,
        ],
    )
    def gather_kernel(table_hbm, idx_hbm, out_hbm, idx_v, rows_v, sem):
        wid = lax.axis_index("s") * NC + lax.axis_index("c")
        pltpu.sync_copy(idx_hbm.at[wid], idx_v)
        base = wid * per_w

        def step(j, carry):
            pltpu.async_copy(table_hbm.at[idx_v.at[j]], rows_v, sem).wait()
            pltpu.sync_copy(rows_v, out_hbm.at[pl.ds(base + j * chunk, chunk)])
            return carry

        lax.fori_loop(0, n_chunks, step, 0)

    return gather_kernel


def _proj_ln_body(g_ref, w_ref, b_ref, gamma_ref, beta_ref, out_ref):
    x = g_ref[...]
    h = lax.dot_general(
        x, w_ref[...],
        dimension_numbers=(((1,), (1,)), ((), ())),
        preferred_element_type=jnp.float32,
    )
    h = h + b_ref[...]
    mu = jnp.mean(h, axis=-1, keepdims=True)
    var = jnp.mean((h - mu) ** 2, axis=-1, keepdims=True)
    y = (h - mu) * lax.rsqrt(var + LN_EPS) * gamma_ref[...] + beta_ref[...]
    y3 = y.reshape(BB, LPAD, PROJ)
    out_ref[...] = y3[:, :50, :]


def _project_ln(g, W, b, gamma, beta, B, L):
    grid = B // BB
    return pl.pallas_call(
        _proj_ln_body,
        grid=(grid,),
        in_specs=[
            pl.BlockSpec((BB * LPAD, EMBED), lambda i: (i, 0)),
            pl.BlockSpec((PROJ, EMBED), lambda i: (0, 0)),
            pl.BlockSpec((1, PROJ), lambda i: (0, 0)),
            pl.BlockSpec((1, PROJ), lambda i: (0, 0)),
            pl.BlockSpec((1, PROJ), lambda i: (0, 0)),
        ],
        out_specs=pl.BlockSpec((BB, L, PROJ), lambda i: (i, 0, 0)),
        out_shape=jax.ShapeDtypeStruct((B, L, PROJ), jnp.float32),
    )(g, W, b.reshape(1, PROJ), gamma.reshape(1, PROJ), beta.reshape(1, PROJ))


def kernel(texts, table, W, b, gamma, beta):
    B, L = texts.shape
    texts_p = jnp.concatenate(
        [texts.astype(jnp.int32), jnp.zeros((B, LPAD - L), jnp.int32)], axis=1)
    n_rows = B * LPAD
    chunk = CH_BATCHES * LPAD
    n_chunks = n_rows // (NW * chunk)
    idx = texts_p.reshape(NW, n_chunks, chunk)
    g = _make_sc_gather(n_rows, n_chunks, chunk)(table, idx)
    return _project_ln(g, W, b, gamma, beta, B, L)


# speedup vs baseline: 1.0047x; 1.0047x over previous
import functools

import jax
import jax.numpy as jnp
from jax import lax
from jax.experimental import pallas as pl
from jax.experimental.pallas import tpu as pltpu
from jax.experimental.pallas import tpu_sc as plsc

VOCAB = 100000
EMBED = 128
PROJ = 512
LN_EPS = 1e-5

NC = 2
NS = 16
NW = NC * NS

LPAD = 56
CHUNK = 128
NBUF = 7
BB = 16


def _make_sc_gather(n_rows, n_chunks):
    per_w = n_chunks * CHUNK
    n_rounds = n_chunks // NBUF
    mesh = plsc.VectorSubcoreMesh(core_axis_name="c", subcore_axis_name="s")

    @functools.partial(
        pl.kernel,
        out_type=jax.ShapeDtypeStruct((n_rows, EMBED), jnp.float32),
        mesh=mesh,
        scratch_types=[
            pltpu.VMEM((n_chunks, CHUNK), jnp.int32),
            [pltpu.VMEM((CHUNK, EMBED), jnp.float32) for _ in range(NBUF)],
            [pltpu.SemaphoreType.DMA for _ in range(NBUF)],
            [pltpu.SemaphoreType.DMA for _ in range(NBUF)],
        ],
    )
    def gather_kernel(table_hbm, idx_hbm, out_hbm, idx_v, bufs, gsems, ssems):
        wid = lax.axis_index("s") * NC + lax.axis_index("c")
        pltpu.sync_copy(idx_hbm.at[wid], idx_v)
        base = wid * per_w

        def gather(c, s):
            pltpu.async_copy(table_hbm.at[idx_v.at[c]], bufs[s], gsems[s])

        def store(c, s):
            pltpu.async_copy(
                bufs[s], out_hbm.at[pl.ds(base + c * CHUNK, CHUNK)], ssems[s])

        for s in range(NBUF):
            gather(s, s)

        def round_body(t, carry):
            for s in range(NBUF):
                c = t * NBUF + s
                pltpu.make_async_copy(table_hbm.at[idx_v.at[c]],
                                      bufs[s], gsems[s]).wait()
                store(c, s)
            for s in range(NBUF):

                @pl.when(t < n_rounds - 1)
                def _():
                    c = t * NBUF + s
                    pltpu.make_async_copy(
                        bufs[s], out_hbm.at[pl.ds(base + c * CHUNK, CHUNK)],
                        ssems[s]).wait()
                    gather(c + NBUF, s)

            return carry

        lax.fori_loop(0, n_rounds, round_body, 0)
        for s in range(NBUF):
            c = (n_rounds - 1) * NBUF + s
            pltpu.make_async_copy(
                bufs[s], out_hbm.at[pl.ds(base + c * CHUNK, CHUNK)],
                ssems[s]).wait()

    return gather_kernel


def _proj_ln_body(g_ref, w_ref, b_ref, gamma_ref, beta_ref, out_ref):
    x = g_ref[...]
    h = lax.dot_general(
        x, w_ref[...],
        dimension_numbers=(((1,), (1,)), ((), ())),
        preferred_element_type=jnp.float32,
    )
    h = h + b_ref[...]
    mu = jnp.mean(h, axis=-1, keepdims=True)
    var = jnp.mean((h - mu) ** 2, axis=-1, keepdims=True)
    y = (h - mu) * lax.rsqrt(var + LN_EPS) * gamma_ref[...] + beta_ref[...]
    y3 = y.reshape(BB, LPAD, PROJ)
    out_ref[...] = y3[:, :50, :]


def _project_ln(g, W, b, gamma, beta, B, L):
    grid = B // BB
    return pl.pallas_call(
        _proj_ln_body,
        grid=(grid,),
        in_specs=[
            pl.BlockSpec((BB * LPAD, EMBED), lambda i: (i, 0)),
            pl.BlockSpec((PROJ, EMBED), lambda i: (0, 0)),
            pl.BlockSpec((1, PROJ), lambda i: (0, 0)),
            pl.BlockSpec((1, PROJ), lambda i: (0, 0)),
            pl.BlockSpec((1, PROJ), lambda i: (0, 0)),
        ],
        out_specs=pl.BlockSpec((BB, L, PROJ), lambda i: (i, 0, 0)),
        out_shape=jax.ShapeDtypeStruct((B, L, PROJ), jnp.float32),
    )(g, W, b.reshape(1, PROJ), gamma.reshape(1, PROJ), beta.reshape(1, PROJ))


def kernel(texts, table, W, b, gamma, beta):
    B, L = texts.shape
    texts_p = jnp.concatenate(
        [texts.astype(jnp.int32), jnp.zeros((B, LPAD - L), jnp.int32)], axis=1)
    n_rows = B * LPAD
    n_chunks = n_rows // (NW * CHUNK)
    idx = texts_p.reshape(NW, n_chunks, CHUNK)
    g = _make_sc_gather(n_rows, n_chunks)(table, idx)
    return _project_ln(g, W, b, gamma, beta, B, L)
